# Initial kernel scaffold; baseline (speedup 1.0000x reference)
import jax
import jax.numpy as jnp
from jax.experimental import pallas as pl
from jax.experimental.pallas import tpu as pltpu

DELTA_VAR = 0.5
DELTA_DIST = 1.5
ALPHA = 1.0
BETA = 1.0
GAMMA = 0.001
KSEG = 16
EPS = 1e-12

TILE = 16384


def _body(emb_ref, mask_ref, out_ref, sums_ref, counts_ref, varsum_ref):
    p = pl.program_id(0)
    b = pl.program_id(1)
    t = pl.program_id(2)
    B = pl.num_programs(1)
    nT = pl.num_programs(2)

    emb = emb_ref[0]
    m = mask_ref[0]
    iota_col = jax.lax.broadcasted_iota(jnp.int32, (KSEG, 1), 0)
    onehot = (m == iota_col).astype(jnp.float32)

    @pl.when(jnp.logical_and(p == 0, t == 0))
    def _():
        sums_ref[b] = jnp.zeros_like(sums_ref[b])
        counts_ref[b] = jnp.zeros_like(counts_ref[b])
        varsum_ref[b] = jnp.zeros_like(varsum_ref[b])

    @pl.when(p == 0)
    def _():
        contrib = jax.lax.dot_general(onehot, emb, (((1,), (1,)), ((), ())),
                                      preferred_element_type=jnp.float32)
        sums_ref[b] += contrib
        counts_ref[b] += jnp.sum(onehot, axis=1)

    @pl.when(p == 1)
    def _():
        counts_b = counts_ref[b]
        safe = jnp.where(counts_b > 0, counts_b, 1.0)
        centers = sums_ref[b] / safe[:, None]
        dots = jax.lax.dot_general(centers, emb, (((1,), (0,)), ((), ())),
                                   preferred_element_type=jnp.float32)
        normsq = jnp.sum(emb * emb, axis=0)
        csq = jnp.sum(centers * centers, axis=1)
        seldot = jnp.sum(onehot * dots, axis=0)
        selcsq = jnp.sum(onehot * csq[:, None], axis=0)
        sq = jnp.maximum(normsq - 2.0 * seldot + selcsq, 0.0)
        d = jnp.sqrt(sq + EPS)
        h = jnp.maximum(d - DELTA_VAR, 0.0)
        varsum_ref[b] += jnp.sum(onehot * (h * h)[None, :], axis=1)

    @pl.when(jnp.logical_and(p == 1,
                             jnp.logical_and(b == B - 1, t == nT - 1)))
    def _():
        kk = jax.lax.broadcasted_iota(jnp.int32, (KSEG, 1), 0)[:, 0]
        lv_acc = jnp.float32(0.0)
        ld_acc = jnp.float32(0.0)
        lr_acc = jnp.float32(0.0)
        vb_acc = jnp.float32(0.0)
        for bi in range(4):
            counts_b = counts_ref[bi]
            present = counts_b > 0
            inst_valid = jnp.logical_and(present, kk > 0)
            n_inst = jnp.sum(inst_valid.astype(jnp.float32))
            safe = jnp.where(present, counts_b, 1.0)
            centers = sums_ref[bi] / safe[:, None]
            var_per = varsum_ref[bi] / safe
            lv = jnp.sum(jnp.where(inst_valid, var_per, 0.0)) / jnp.maximum(n_inst, 1.0)
            csq = jnp.sum(centers * centers, axis=1)
            gram = jax.lax.dot_general(centers, centers, (((1,), (1,)), ((), ())),
                                       preferred_element_type=jnp.float32)
            sq_pair = jnp.maximum(csq[:, None] + csq[None, :] - 2.0 * gram, 0.0)
            pm = inst_valid[:, None] & inst_valid[None, :] & (kk[:, None] < kk[None, :])
            pair_d = jnp.sqrt(jnp.where(pm, sq_pair, 1.0))
            hd = jnp.maximum(2.0 * DELTA_DIST - pair_d, 0.0) ** 2
            n_pairs = jnp.sum(pm.astype(jnp.float32))
            ld = jnp.sum(jnp.where(pm, hd, 0.0)) / jnp.maximum(n_pairs, 1.0)
            c_norm = jnp.sqrt(jnp.where(inst_valid, csq, 1.0))
            lr = jnp.sum(jnp.where(inst_valid, c_norm, 0.0)) / jnp.maximum(n_inst, 1.0)
            validb = (n_inst > 0).astype(jnp.float32)
            lv_acc += lv * validb
            ld_acc += ld * validb
            lr_acc += lr * validb
            vb_acc += validb
        denom = jnp.maximum(vb_acc, 1.0)
        lvt = lv_acc / denom
        ldt = ld_acc / denom
        lrt = lr_acc / denom
        total = ALPHA * lvt + BETA * ldt + GAMMA * lrt
        row = jax.lax.broadcasted_iota(jnp.int32, (8, 128), 0)
        col = jax.lax.broadcasted_iota(jnp.int32, (8, 128), 1)
        vals = jnp.where(col == 0, total,
               jnp.where(col == 1, lvt,
               jnp.where(col == 2, ldt, lrt)))
        out_ref[...] = jnp.where(row == 0, vals, 0.0)


def kernel(embedding, instance_mask):
    if instance_mask.ndim == 4:
        instance_mask = instance_mask[:, 0]
    B, E, H, W = embedding.shape
    P = H * W
    emb3 = embedding.reshape(B, E, P)
    mask3 = instance_mask.reshape(B, 1, P)
    nT = P // TILE

    out = pl.pallas_call(
        _body,
        grid=(2, B, nT),
        in_specs=[
            pl.BlockSpec((1, E, TILE), lambda p, b, t: (b, 0, t)),
            pl.BlockSpec((1, 1, TILE), lambda p, b, t: (b, 0, t)),
        ],
        out_specs=pl.BlockSpec((8, 128), lambda p, b, t: (0, 0)),
        out_shape=jax.ShapeDtypeStruct((8, 128), jnp.float32),
        scratch_shapes=[
            pltpu.VMEM((B, KSEG, E), jnp.float32),
            pltpu.VMEM((B, KSEG), jnp.float32),
            pltpu.VMEM((B, KSEG), jnp.float32),
        ],
        compiler_params=pltpu.CompilerParams(
            dimension_semantics=("arbitrary", "arbitrary", "arbitrary"),
        ),
    )(emb3, mask3)
    return (out[0, 0], out[0, 1], out[0, 2], out[0, 3])


# speedup vs baseline: 50.7538x; 50.7538x over previous
import jax
import jax.numpy as jnp
from jax.experimental import pallas as pl
from jax.experimental.pallas import tpu as pltpu

DELTA_VAR = 0.5
DELTA_DIST = 1.5
ALPHA = 1.0
BETA = 1.0
GAMMA = 0.001
KSEG = 16
EPS = 1e-12

TILE = 16384


def _body(emb_ref, mask_ref, out_ref, sums_ref, counts_ref, varsum_ref):
    p = pl.program_id(0)
    b = pl.program_id(1)
    t = pl.program_id(2)
    B = pl.num_programs(1)
    nT = pl.num_programs(2)

    emb = emb_ref[0]
    m = mask_ref[0]
    iota_col = jax.lax.broadcasted_iota(jnp.int32, (KSEG, 1), 0)
    onehot = (m == iota_col).astype(jnp.float32)

    @pl.when(jnp.logical_and(p == 0, t == 0))
    def _():
        sums_ref[b] = jnp.zeros_like(sums_ref[b])
        counts_ref[b] = jnp.zeros_like(counts_ref[b])
        varsum_ref[b] = jnp.zeros_like(varsum_ref[b])

    @pl.when(p == 0)
    def _():
        contrib = jax.lax.dot_general(onehot, emb, (((1,), (1,)), ((), ())),
                                      preferred_element_type=jnp.float32)
        sums_ref[b] += contrib
        counts_ref[b] += jnp.sum(onehot, axis=1, keepdims=True)

    @pl.when(p == 1)
    def _():
        counts_b = counts_ref[b]
        safe = jnp.where(counts_b > 0, counts_b, 1.0)
        centers = sums_ref[b] / safe
        dots = jax.lax.dot_general(centers, emb, (((1,), (0,)), ((), ())),
                                   preferred_element_type=jnp.float32)
        normsq = jnp.sum(emb * emb, axis=0)
        csq = jnp.sum(centers * centers, axis=1, keepdims=True)
        seldot = jnp.sum(onehot * dots, axis=0)
        selcsq = jnp.sum(onehot * csq, axis=0)
        sq = jnp.maximum(normsq - 2.0 * seldot + selcsq, 0.0)
        d = jnp.sqrt(sq + EPS)
        h = jnp.maximum(d - DELTA_VAR, 0.0)
        varsum_ref[b] += jnp.sum(onehot * (h * h)[None, :], axis=1,
                                 keepdims=True)

    @pl.when(jnp.logical_and(p == 1,
                             jnp.logical_and(b == B - 1, t == nT - 1)))
    def _():
        kk_col = jax.lax.broadcasted_iota(jnp.int32, (KSEG, 1), 0)
        kk_row = jax.lax.broadcasted_iota(jnp.int32, (1, KSEG), 1)
        kk_sq_r = jax.lax.broadcasted_iota(jnp.int32, (KSEG, KSEG), 1)
        kk_sq_c = jax.lax.broadcasted_iota(jnp.int32, (KSEG, KSEG), 0)
        eye = (kk_col == kk_row).astype(jnp.float32)
        lv_acc = jnp.float32(0.0)
        ld_acc = jnp.float32(0.0)
        lr_acc = jnp.float32(0.0)
        vb_acc = jnp.float32(0.0)
        for bi in range(4):
            counts_b = counts_ref[bi]
            valid_col = jnp.logical_and(counts_b > 0, kk_col > 0)
            vcf = valid_col.astype(jnp.float32)
            n_inst = jnp.sum(vcf)
            safe = jnp.where(counts_b > 0, counts_b, 1.0)
            centers = sums_ref[bi] / safe
            var_per = varsum_ref[bi] / safe
            lv = jnp.sum(jnp.where(valid_col, var_per, 0.0)) / jnp.maximum(n_inst, 1.0)
            csq = jnp.sum(centers * centers, axis=1, keepdims=True)
            gram = jax.lax.dot_general(centers, centers, (((1,), (1,)), ((), ())),
                                       preferred_element_type=jnp.float32)
            csq_row = jnp.sum(eye * gram, axis=0, keepdims=True)
            sq_pair = jnp.maximum(csq + csq_row - 2.0 * gram, 0.0)
            outer = jax.lax.dot_general(vcf, vcf, (((1,), (1,)), ((), ())),
                                        preferred_element_type=jnp.float32)
            pm = jnp.logical_and(outer > 0.5, kk_sq_c < kk_sq_r)
            pair_d = jnp.sqrt(jnp.where(pm, sq_pair, 1.0))
            hd = jnp.maximum(2.0 * DELTA_DIST - pair_d, 0.0) ** 2
            n_pairs = jnp.sum(pm.astype(jnp.float32))
            ld = jnp.sum(jnp.where(pm, hd, 0.0)) / jnp.maximum(n_pairs, 1.0)
            c_norm = jnp.sqrt(jnp.where(valid_col, csq, 1.0))
            lr = jnp.sum(jnp.where(valid_col, c_norm, 0.0)) / jnp.maximum(n_inst, 1.0)
            validb = (n_inst > 0).astype(jnp.float32)
            lv_acc += lv * validb
            ld_acc += ld * validb
            lr_acc += lr * validb
            vb_acc += validb
        denom = jnp.maximum(vb_acc, 1.0)
        lvt = lv_acc / denom
        ldt = ld_acc / denom
        lrt = lr_acc / denom
        total = ALPHA * lvt + BETA * ldt + GAMMA * lrt
        row = jax.lax.broadcasted_iota(jnp.int32, (8, 128), 0)
        col = jax.lax.broadcasted_iota(jnp.int32, (8, 128), 1)
        vals = jnp.where(col == 0, total,
               jnp.where(col == 1, lvt,
               jnp.where(col == 2, ldt, lrt)))
        out_ref[...] = jnp.where(row == 0, vals, 0.0)


def kernel(embedding, instance_mask):
    if instance_mask.ndim == 4:
        instance_mask = instance_mask[:, 0]
    B, E, H, W = embedding.shape
    P = H * W
    emb3 = embedding.reshape(B, E, P)
    mask3 = instance_mask.reshape(B, 1, P)
    nT = P // TILE

    out = pl.pallas_call(
        _body,
        grid=(2, B, nT),
        in_specs=[
            pl.BlockSpec((1, E, TILE), lambda p, b, t: (b, 0, t)),
            pl.BlockSpec((1, 1, TILE), lambda p, b, t: (b, 0, t)),
        ],
        out_specs=pl.BlockSpec((8, 128), lambda p, b, t: (0, 0)),
        out_shape=jax.ShapeDtypeStruct((8, 128), jnp.float32),
        scratch_shapes=[
            pltpu.VMEM((B, KSEG, E), jnp.float32),
            pltpu.VMEM((B, KSEG, 1), jnp.float32),
            pltpu.VMEM((B, KSEG, 1), jnp.float32),
        ],
        compiler_params=pltpu.CompilerParams(
            dimension_semantics=("arbitrary", "arbitrary", "arbitrary"),
        ),
    )(emb3, mask3)
    return (out[0, 0], out[0, 1], out[0, 2], out[0, 3])
